# Initial kernel scaffold; baseline (speedup 1.0000x reference)
import jax
import jax.numpy as jnp
from jax.experimental import pallas as pl


def kernel(x, edge_index, edge_weight, W, b, gamma, beta):
    raise NotImplementedError("write your pallas kernel here")


# speedup vs baseline: 2.4644x; 2.4644x over previous
import functools

import jax
import jax.numpy as jnp
from jax import lax
from jax.experimental import pallas as pl
from jax.experimental.pallas import tpu as pltpu
from jax.experimental.pallas import tpu_sc as plsc

N = 10000
D = 256
H = 128
E = 160000
CH = 128
TILES = 16
NCH = 79
EPT = NCH * CH
E_PAD = TILES * EPT
N_PAD = 10240
NPT = N_PAD // TILES
LANES = 16


def _sc_diffuse(xflat, src2, dstp, wp, zrows):
    mesh = plsc.VectorSubcoreMesh(core_axis_name="c", subcore_axis_name="s")

    @functools.partial(
        pl.kernel,
        out_type=jax.ShapeDtypeStruct((2 * N_PAD, H), jnp.float32),
        mesh=mesh,
        scratch_types=[
            pltpu.VMEM((CH,), jnp.int32),
            pltpu.VMEM((CH,), jnp.int32),
            pltpu.VMEM((CH + LANES,), jnp.float32),
            pltpu.VMEM((CH, H), jnp.float32),
            pltpu.VMEM_SHARED((N_PAD, H), jnp.float32),
            pltpu.SemaphoreType.DMA,
        ],
    )
    def k(x_hbm, src_hbm, dst_hbm, w_hbm, z_hbm, out_hbm,
          src_v, dst_v, w_v, rows_v, acc_sh, sem):
        c = lax.axis_index("c")
        s = lax.axis_index("s")

        pltpu.sync_copy(z_hbm, acc_sh.at[pl.ds(s * NPT, NPT)])
        plsc.subcore_barrier()

        ebase = c * E_PAD + s * EPT

        def chunk_body(kk, carry):
            off = ebase + kk * CH
            doff = s * EPT + kk * CH
            pltpu.sync_copy(src_hbm.at[pl.ds(off, CH)], src_v)
            pltpu.sync_copy(dst_hbm.at[pl.ds(doff, CH)], dst_v)
            pltpu.sync_copy(w_hbm.at[pl.ds(doff, CH)], w_v.at[pl.ds(0, CH)])
            pltpu.async_copy(x_hbm.at[src_v], rows_v, sem).wait()

            def edge_body(e, carry2):
                w = w_v[pl.ds(e, LANES)][0]
                for j in range(H // LANES):
                    sl = pl.ds(j * LANES, LANES)
                    rows_v[e, sl] = rows_v[e, sl] * w
                return carry2

            lax.fori_loop(0, CH, edge_body, 0, unroll=2)
            pltpu.sync_copy(rows_v, acc_sh.at[dst_v], add=True)
            return carry

        lax.fori_loop(0, NCH, chunk_body, 0)
        plsc.subcore_barrier()

        pltpu.sync_copy(acc_sh.at[pl.ds(s * NPT, NPT)],
                        out_hbm.at[pl.ds(c * N_PAD + s * NPT, NPT)])

    return k(xflat, src2, dstp, wp, zrows)


def _mm_stats_body(al_ref, ar_ref, w1_ref, w2_ref, b_ref, z_ref, st_ref, acc):
    i = pl.program_id(0)
    z = jnp.dot(al_ref[...], w1_ref[...], preferred_element_type=jnp.float32)
    z = z + jnp.dot(ar_ref[...], w2_ref[...], preferred_element_type=jnp.float32)
    z = z + b_ref[0:1, :]
    z_ref[...] = z

    @pl.when(i == 0)
    def _():
        acc[...] = jnp.zeros((8, D), jnp.float32)

    acc[0:1, :] += jnp.sum(z, axis=0, keepdims=True)
    acc[1:2, :] += jnp.sum(z * z, axis=0, keepdims=True)

    @pl.when(i == pl.num_programs(0) - 1)
    def _():
        st_ref[...] = acc[...]


def _bn_body(z_ref, st_ref, g_ref, be_ref, o_ref):
    inv_n = 1.0 / N
    mean = st_ref[0:1, :] * inv_n
    var = st_ref[1:2, :] * inv_n - mean * mean
    inv = lax.rsqrt(var + 1e-5)
    scale = g_ref[0:1, :] * inv
    shift = be_ref[0:1, :] - mean * scale
    y = z_ref[...] * scale + shift
    o_ref[...] = jnp.where(y >= 0, y, 0.01 * y)


_BLK = 400
_GRID = N // _BLK


def _tc_finish(aggl, aggr, W, b, gamma, beta, interpret=False):
    w1 = W[:H, :]
    w2 = W[H:, :]
    b8 = jnp.broadcast_to(b.reshape(1, D), (8, D))
    g8 = jnp.broadcast_to(gamma.reshape(1, D), (8, D))
    be8 = jnp.broadcast_to(beta.reshape(1, D), (8, D))

    z, stats = pl.pallas_call(
        _mm_stats_body,
        grid=(_GRID,),
        in_specs=[
            pl.BlockSpec((_BLK, H), lambda i: (i, 0)),
            pl.BlockSpec((_BLK, H), lambda i: (i, 0)),
            pl.BlockSpec((H, D), lambda i: (0, 0)),
            pl.BlockSpec((H, D), lambda i: (0, 0)),
            pl.BlockSpec((8, D), lambda i: (0, 0)),
        ],
        out_specs=[
            pl.BlockSpec((_BLK, D), lambda i: (i, 0)),
            pl.BlockSpec((8, D), lambda i: (0, 0)),
        ],
        out_shape=[
            jax.ShapeDtypeStruct((N, D), jnp.float32),
            jax.ShapeDtypeStruct((8, D), jnp.float32),
        ],
        scratch_shapes=[pltpu.VMEM((8, D), jnp.float32)],
        interpret=interpret,
    )(aggl, aggr, w1, w2, b8)

    out = pl.pallas_call(
        _bn_body,
        grid=(_GRID,),
        in_specs=[
            pl.BlockSpec((_BLK, D), lambda i: (i, 0)),
            pl.BlockSpec((8, D), lambda i: (0, 0)),
            pl.BlockSpec((8, D), lambda i: (0, 0)),
            pl.BlockSpec((8, D), lambda i: (0, 0)),
        ],
        out_specs=pl.BlockSpec((_BLK, D), lambda i: (i, 0)),
        out_shape=jax.ShapeDtypeStruct((N, D), jnp.float32),
        interpret=interpret,
    )(z, stats, g8, be8)
    return out


def kernel(x, edge_index, edge_weight, W, b, gamma, beta):
    xflat = jnp.concatenate([x[:, :H], x[:, H:]], axis=0)
    pad = E_PAD - E
    src = jnp.pad(edge_index[0], (0, pad))
    dstp = jnp.pad(edge_index[1], (0, pad))
    wp = jnp.pad(edge_weight, (0, pad))
    src2 = jnp.concatenate([src, src + N])
    zrows = jnp.zeros((NPT, H), jnp.float32)

    agg = _sc_diffuse(xflat, src2, dstp, wp, zrows)
    aggl = agg[:N, :]
    aggr = agg[N_PAD:N_PAD + N, :]
    return _tc_finish(aggl, aggr, W, b, gamma, beta)


# speedup vs baseline: 3.3496x; 1.3592x over previous
import functools

import jax
import jax.numpy as jnp
from jax import lax
from jax.experimental import pallas as pl
from jax.experimental.pallas import tpu as pltpu
from jax.experimental.pallas import tpu_sc as plsc

N = 10000
D = 256
H = 128
E = 160000
CH = 80
TILES = 16
NCH = 128
EPT = NCH * CH
E_PAD = TILES * EPT
N_PAD = 10112
NPT = N_PAD // TILES
LANES = 16
NBUF = 4
IW = 2 * CH


def _sc_diffuse(xflat, src_flat, w_flat, dst_flat, zrows):
    mesh = plsc.VectorSubcoreMesh(core_axis_name="c", subcore_axis_name="s")
    n_iter = NCH // NBUF

    @functools.partial(
        pl.kernel,
        out_type=jax.ShapeDtypeStruct((2 * N_PAD, H), jnp.float32),
        mesh=mesh,
        scratch_types=[
            [pltpu.VMEM((CH,), jnp.int32) for _ in range(NBUF)],
            [pltpu.VMEM((CH,), jnp.float32) for _ in range(NBUF)],
            [pltpu.VMEM((CH,), jnp.int32) for _ in range(NBUF)],
            [pltpu.VMEM((CH, H), jnp.float32) for _ in range(NBUF)],
            pltpu.VMEM_SHARED((N_PAD, H), jnp.float32),
            [pltpu.SemaphoreType.DMA for _ in range(NBUF)],
            [pltpu.SemaphoreType.DMA for _ in range(NBUF)],
            [pltpu.SemaphoreType.DMA for _ in range(NBUF)],
            [pltpu.SemaphoreType.DMA for _ in range(NBUF)],
            [pltpu.SemaphoreType.DMA for _ in range(NBUF)],
        ],
    )
    def k(x_hbm, src_hbm, w_hbm, dst_hbm, z_hbm, out_hbm,
          sbufs, wbufs, dbufs, bufs, acc_sh, isems, wsems, dsems, gsems,
          ssems):
        c = lax.axis_index("c")
        s = lax.axis_index("s")
        ibase = c * E_PAD + s * EPT
        dbase = s * EPT

        def iw_dma(k_, j):
            pltpu.async_copy(src_hbm.at[pl.ds(ibase + k_ * CH, CH)],
                             sbufs[j], isems[j])
            pltpu.async_copy(w_hbm.at[pl.ds(dbase + k_ * CH, CH)],
                             wbufs[j], wsems[j])

        def iw_wait(k_, j):
            pltpu.make_async_copy(src_hbm.at[pl.ds(ibase + k_ * CH, CH)],
                                  sbufs[j], isems[j]).wait()

        def w_wait(k_, j):
            pltpu.make_async_copy(w_hbm.at[pl.ds(dbase + k_ * CH, CH)],
                                  wbufs[j], wsems[j]).wait()

        def dst_dma(k_, j):
            pltpu.async_copy(dst_hbm.at[pl.ds(dbase + k_ * CH, CH)],
                             dbufs[j], dsems[j])

        def dst_wait(k_, j):
            pltpu.make_async_copy(dst_hbm.at[pl.ds(dbase + k_ * CH, CH)],
                                  dbufs[j], dsems[j]).wait()

        def gather(k_, j):
            pltpu.async_copy(x_hbm.at[sbufs[j]], bufs[j], gsems[j])

        def gather_wait(k_, j):
            pltpu.make_async_copy(x_hbm.at[sbufs[j]], bufs[j],
                                  gsems[j]).wait()

        def scatter(k_, j):
            pltpu.async_copy(bufs[j], acc_sh.at[dbufs[j]], ssems[j], add=True)

        def scatter_wait(k_, j):
            pltpu.make_async_copy(bufs[j], acc_sh.at[dbufs[j]],
                                  ssems[j]).wait()

        def multiply(j):
            buf = bufs[j]
            wb = wbufs[j]

            def grp_body(g, carry2):
                wv = wb[pl.ds(g * LANES, LANES)]
                for t in range(LANES):
                    e = g * LANES + t
                    w = wv[t]
                    for u in range(H // LANES):
                        sl = pl.ds(u * LANES, LANES)
                        buf[e, sl] = buf[e, sl] * w
                return carry2

            lax.fori_loop(0, CH // LANES, grp_body, 0)

        pltpu.sync_copy(z_hbm, acc_sh.at[pl.ds(s * NPT, NPT)])
        plsc.subcore_barrier()

        iw_dma(0, 0)
        iw_dma(1, 1)
        dst_dma(0, 0)
        dst_dma(1, 1)
        iw_wait(0, 0)
        gather(0, 0)
        iw_wait(1, 1)
        gather(1, 1)
        iw_dma(2, 2)
        iw_dma(3, 3)

        def loop_body(m, carry):
            for r in range(NBUF):
                k_ = NBUF * m + r
                jn = (r + 2) % NBUF

                def _head():
                    scatter_wait(k_ - 2, jn)

                def _pref():
                    iw_wait(k_ + 2, jn)
                    gather(k_ + 2, jn)
                    dst_dma(k_ + 2, jn)

                if r in (0, 1):
                    @pl.when(m >= 1)
                    def _():
                        _head()

                    _pref()
                else:
                    _head()

                    @pl.when(m < n_iter - 1)
                    def _():
                        _pref()

                gather_wait(k_, r)
                w_wait(k_, r)
                multiply(r)

                @pl.when(m < n_iter - 1)
                def _():
                    iw_dma(k_ + NBUF, r)

                dst_wait(k_, r)
                scatter(k_, r)
            return carry

        lax.fori_loop(0, n_iter, loop_body, 0)
        scatter_wait(NCH - 2, (NCH - 2) % NBUF)
        scatter_wait(NCH - 1, (NCH - 1) % NBUF)
        plsc.subcore_barrier()

        pltpu.sync_copy(acc_sh.at[pl.ds(s * NPT, NPT)],
                        out_hbm.at[pl.ds(c * N_PAD + s * NPT, NPT)])

    return k(xflat, src_flat, w_flat, dst_flat, zrows)


def _mm_stats_body(al_ref, ar_ref, w1_ref, w2_ref, b_ref, z_ref, st_ref, acc):
    i = pl.program_id(0)
    z = jnp.dot(al_ref[...], w1_ref[...], preferred_element_type=jnp.float32)
    z = z + jnp.dot(ar_ref[...], w2_ref[...], preferred_element_type=jnp.float32)
    z = z + b_ref[0:1, :]
    z_ref[...] = z

    @pl.when(i == 0)
    def _():
        acc[...] = jnp.zeros((8, D), jnp.float32)

    acc[0:1, :] += jnp.sum(z, axis=0, keepdims=True)
    acc[1:2, :] += jnp.sum(z * z, axis=0, keepdims=True)

    @pl.when(i == pl.num_programs(0) - 1)
    def _():
        st_ref[...] = acc[...]


def _bn_body(z_ref, st_ref, g_ref, be_ref, o_ref):
    inv_n = 1.0 / N
    mean = st_ref[0:1, :] * inv_n
    var = st_ref[1:2, :] * inv_n - mean * mean
    inv = lax.rsqrt(var + 1e-5)
    scale = g_ref[0:1, :] * inv
    shift = be_ref[0:1, :] - mean * scale
    y = z_ref[...] * scale + shift
    o_ref[...] = jnp.where(y >= 0, y, 0.01 * y)


_BLK = 400
_GRID = N // _BLK


def _tc_finish(aggl, aggr, W, b, gamma, beta, interpret=False):
    w1 = W[:H, :]
    w2 = W[H:, :]
    b8 = jnp.broadcast_to(b.reshape(1, D), (8, D))
    g8 = jnp.broadcast_to(gamma.reshape(1, D), (8, D))
    be8 = jnp.broadcast_to(beta.reshape(1, D), (8, D))

    z, stats = pl.pallas_call(
        _mm_stats_body,
        grid=(_GRID,),
        in_specs=[
            pl.BlockSpec((_BLK, H), lambda i: (i, 0)),
            pl.BlockSpec((_BLK, H), lambda i: (i, 0)),
            pl.BlockSpec((H, D), lambda i: (0, 0)),
            pl.BlockSpec((H, D), lambda i: (0, 0)),
            pl.BlockSpec((8, D), lambda i: (0, 0)),
        ],
        out_specs=[
            pl.BlockSpec((_BLK, D), lambda i: (i, 0)),
            pl.BlockSpec((8, D), lambda i: (0, 0)),
        ],
        out_shape=[
            jax.ShapeDtypeStruct((N, D), jnp.float32),
            jax.ShapeDtypeStruct((8, D), jnp.float32),
        ],
        scratch_shapes=[pltpu.VMEM((8, D), jnp.float32)],
        interpret=interpret,
    )(aggl, aggr, w1, w2, b8)

    out = pl.pallas_call(
        _bn_body,
        grid=(_GRID,),
        in_specs=[
            pl.BlockSpec((_BLK, D), lambda i: (i, 0)),
            pl.BlockSpec((8, D), lambda i: (0, 0)),
            pl.BlockSpec((8, D), lambda i: (0, 0)),
            pl.BlockSpec((8, D), lambda i: (0, 0)),
        ],
        out_specs=pl.BlockSpec((_BLK, D), lambda i: (i, 0)),
        out_shape=jax.ShapeDtypeStruct((N, D), jnp.float32),
        interpret=interpret,
    )(z, stats, g8, be8)
    return out


def kernel(x, edge_index, edge_weight, W, b, gamma, beta):
    xflat = jnp.concatenate([x[:, :H], x[:, H:]], axis=0)
    pad = E_PAD - E
    srcp = jnp.pad(edge_index[0], (0, pad))
    dstp = jnp.pad(edge_index[1], (0, pad))
    wp = jnp.pad(edge_weight, (0, pad))
    src2 = jnp.concatenate([srcp, srcp + N])
    zrows = jnp.zeros((NPT, H), jnp.float32)

    agg = _sc_diffuse(xflat, src2, wp, dstp, zrows)
    aggl = agg[:N, :]
    aggr = agg[N_PAD:N_PAD + N, :]
    return _tc_finish(aggl, aggr, W, b, gamma, beta)


# speedup vs baseline: 3.4397x; 1.0269x over previous
import functools

import jax
import jax.numpy as jnp
from jax import lax
from jax.experimental import pallas as pl
from jax.experimental.pallas import tpu as pltpu
from jax.experimental.pallas import tpu_sc as plsc

N = 10000
D = 256
H = 128
E = 160000
CH = 80
TILES = 16
NCH = 128
EPT = NCH * CH
E_PAD = TILES * EPT
N_PAD = 10112
NPT = N_PAD // TILES
LANES = 16
NBUF = 4
IW = 2 * CH


def _sc_diffuse(xflat, src_flat, w_flat, dst_flat, zrows):
    mesh = plsc.VectorSubcoreMesh(core_axis_name="c", subcore_axis_name="s")
    n_iter = NCH // NBUF

    @functools.partial(
        pl.kernel,
        out_type=jax.ShapeDtypeStruct((2 * N_PAD, H), jnp.float32),
        mesh=mesh,
        scratch_types=[
            [pltpu.VMEM((CH,), jnp.int32) for _ in range(NBUF)],
            [pltpu.VMEM((CH,), jnp.float32) for _ in range(NBUF)],
            [pltpu.VMEM((CH,), jnp.int32) for _ in range(NBUF)],
            [pltpu.VMEM((CH, H), jnp.float32) for _ in range(NBUF)],
            pltpu.VMEM_SHARED((N_PAD, H), jnp.float32),
            [pltpu.SemaphoreType.DMA for _ in range(NBUF)],
            [pltpu.SemaphoreType.DMA for _ in range(NBUF)],
            [pltpu.SemaphoreType.DMA for _ in range(NBUF)],
            [pltpu.SemaphoreType.DMA for _ in range(NBUF)],
            [pltpu.SemaphoreType.DMA for _ in range(NBUF)],
        ],
    )
    def k(x_hbm, src_hbm, w_hbm, dst_hbm, z_hbm, out_hbm,
          sbufs, wbufs, dbufs, bufs, acc_sh, isems, wsems, dsems, gsems,
          ssems):
        c = lax.axis_index("c")
        s = lax.axis_index("s")
        ibase = s * EPT
        dbase = s * EPT

        def iw_dma(k_, j):
            pltpu.async_copy(src_hbm.at[pl.ds(ibase + k_ * CH, CH)],
                             sbufs[j], isems[j])
            pltpu.async_copy(w_hbm.at[pl.ds(dbase + k_ * CH, CH)],
                             wbufs[j], wsems[j])

        def iw_wait(k_, j):
            pltpu.make_async_copy(src_hbm.at[pl.ds(ibase + k_ * CH, CH)],
                                  sbufs[j], isems[j]).wait()

        def xform(j):
            for g in range(CH // LANES):
                sl = pl.ds(g * LANES, LANES)
                v = sbufs[j][sl]
                sbufs[j][sl] = v + v + c

        def w_wait(k_, j):
            pltpu.make_async_copy(w_hbm.at[pl.ds(dbase + k_ * CH, CH)],
                                  wbufs[j], wsems[j]).wait()

        def dst_dma(k_, j):
            pltpu.async_copy(dst_hbm.at[pl.ds(dbase + k_ * CH, CH)],
                             dbufs[j], dsems[j])

        def dst_wait(k_, j):
            pltpu.make_async_copy(dst_hbm.at[pl.ds(dbase + k_ * CH, CH)],
                                  dbufs[j], dsems[j]).wait()

        def gather(k_, j):
            pltpu.async_copy(x_hbm.at[sbufs[j]], bufs[j], gsems[j])

        def gather_wait(k_, j):
            pltpu.make_async_copy(x_hbm.at[sbufs[j]], bufs[j],
                                  gsems[j]).wait()

        def scatter(k_, j):
            pltpu.async_copy(bufs[j], acc_sh.at[dbufs[j]], ssems[j], add=True)

        def scatter_wait(k_, j):
            pltpu.make_async_copy(bufs[j], acc_sh.at[dbufs[j]],
                                  ssems[j]).wait()

        def multiply(j):
            buf = bufs[j]
            wb = wbufs[j]

            def grp_body(g, carry2):
                wv = wb[pl.ds(g * LANES, LANES)]
                for t in range(LANES):
                    e = g * LANES + t
                    w = wv[t]
                    for u in range(H // LANES):
                        sl = pl.ds(u * LANES, LANES)
                        buf[e, sl] = buf[e, sl] * w
                return carry2

            lax.fori_loop(0, CH // LANES, grp_body, 0)

        pltpu.sync_copy(z_hbm, acc_sh.at[pl.ds(s * NPT, NPT)])
        plsc.subcore_barrier()

        iw_dma(0, 0)
        iw_dma(1, 1)
        dst_dma(0, 0)
        dst_dma(1, 1)
        iw_wait(0, 0)
        xform(0)
        gather(0, 0)
        iw_wait(1, 1)
        xform(1)
        gather(1, 1)
        iw_dma(2, 2)
        iw_dma(3, 3)

        def loop_body(m, carry):
            for r in range(NBUF):
                k_ = NBUF * m + r
                jn = (r + 2) % NBUF

                def _head():
                    scatter_wait(k_ - 2, jn)

                def _pref():
                    iw_wait(k_ + 2, jn)
                    xform(jn)
                    gather(k_ + 2, jn)
                    dst_dma(k_ + 2, jn)

                if r in (0, 1):
                    @pl.when(m >= 1)
                    def _():
                        _head()

                    _pref()
                else:
                    _head()

                    @pl.when(m < n_iter - 1)
                    def _():
                        _pref()

                gather_wait(k_, r)
                w_wait(k_, r)
                multiply(r)

                @pl.when(m < n_iter - 1)
                def _():
                    iw_dma(k_ + NBUF, r)

                dst_wait(k_, r)
                scatter(k_, r)
            return carry

        lax.fori_loop(0, n_iter, loop_body, 0)
        scatter_wait(NCH - 2, (NCH - 2) % NBUF)
        scatter_wait(NCH - 1, (NCH - 1) % NBUF)
        plsc.subcore_barrier()

        pltpu.sync_copy(acc_sh.at[pl.ds(s * NPT, NPT)],
                        out_hbm.at[pl.ds(c * N_PAD + s * NPT, NPT)])

    return k(xflat, src_flat, w_flat, dst_flat, zrows)


def _mm_stats_body(al_ref, ar_ref, w1_ref, w2_ref, b_ref, z_ref, st_ref, acc):
    i = pl.program_id(0)
    z = jnp.dot(al_ref[...], w1_ref[...], preferred_element_type=jnp.float32)
    z = z + jnp.dot(ar_ref[...], w2_ref[...], preferred_element_type=jnp.float32)
    z = z + b_ref[0:1, :]
    z_ref[...] = z

    @pl.when(i == 0)
    def _():
        acc[...] = jnp.zeros((8, D), jnp.float32)

    acc[0:1, :] += jnp.sum(z, axis=0, keepdims=True)
    acc[1:2, :] += jnp.sum(z * z, axis=0, keepdims=True)

    @pl.when(i == pl.num_programs(0) - 1)
    def _():
        st_ref[...] = acc[...]


def _bn_body(z_ref, st_ref, g_ref, be_ref, o_ref):
    inv_n = 1.0 / N
    mean = st_ref[0:1, :] * inv_n
    var = st_ref[1:2, :] * inv_n - mean * mean
    inv = lax.rsqrt(var + 1e-5)
    scale = g_ref[0:1, :] * inv
    shift = be_ref[0:1, :] - mean * scale
    y = z_ref[...] * scale + shift
    o_ref[...] = jnp.where(y >= 0, y, 0.01 * y)


_BLK = 400
_GRID = N // _BLK


def _tc_finish(aggl, aggr, W, b, gamma, beta, interpret=False):
    w1 = W[:H, :]
    w2 = W[H:, :]
    b8 = jnp.broadcast_to(b.reshape(1, D), (8, D))
    g8 = jnp.broadcast_to(gamma.reshape(1, D), (8, D))
    be8 = jnp.broadcast_to(beta.reshape(1, D), (8, D))

    z, stats = pl.pallas_call(
        _mm_stats_body,
        grid=(_GRID,),
        in_specs=[
            pl.BlockSpec((_BLK, H), lambda i: (i, 0)),
            pl.BlockSpec((_BLK, H), lambda i: (i, 0)),
            pl.BlockSpec((H, D), lambda i: (0, 0)),
            pl.BlockSpec((H, D), lambda i: (0, 0)),
            pl.BlockSpec((8, D), lambda i: (0, 0)),
        ],
        out_specs=[
            pl.BlockSpec((_BLK, D), lambda i: (i, 0)),
            pl.BlockSpec((8, D), lambda i: (0, 0)),
        ],
        out_shape=[
            jax.ShapeDtypeStruct((N, D), jnp.float32),
            jax.ShapeDtypeStruct((8, D), jnp.float32),
        ],
        scratch_shapes=[pltpu.VMEM((8, D), jnp.float32)],
        interpret=interpret,
    )(aggl, aggr, w1, w2, b8)

    out = pl.pallas_call(
        _bn_body,
        grid=(_GRID,),
        in_specs=[
            pl.BlockSpec((_BLK, D), lambda i: (i, 0)),
            pl.BlockSpec((8, D), lambda i: (0, 0)),
            pl.BlockSpec((8, D), lambda i: (0, 0)),
            pl.BlockSpec((8, D), lambda i: (0, 0)),
        ],
        out_specs=pl.BlockSpec((_BLK, D), lambda i: (i, 0)),
        out_shape=jax.ShapeDtypeStruct((N, D), jnp.float32),
        interpret=interpret,
    )(z, stats, g8, be8)
    return out


def kernel(x, edge_index, edge_weight, W, b, gamma, beta):
    xflat = x.reshape(2 * N, H)
    pad = E_PAD - E
    srcp = jnp.pad(edge_index[0], (0, pad))
    dstp = jnp.pad(edge_index[1], (0, pad))
    wp = jnp.pad(edge_weight, (0, pad))
    zrows = jnp.zeros((NPT, H), jnp.float32)

    agg = _sc_diffuse(xflat, srcp, wp, dstp, zrows)
    aggl = agg[:N, :]
    aggr = agg[N_PAD:N_PAD + N, :]
    return _tc_finish(aggl, aggr, W, b, gamma, beta)


# speedup vs baseline: 3.4472x; 1.0022x over previous
import functools

import jax
import jax.numpy as jnp
from jax import lax
from jax.experimental import pallas as pl
from jax.experimental.pallas import tpu as pltpu
from jax.experimental.pallas import tpu_sc as plsc

N = 10000
D = 256
H = 128
E = 160000
CH = 80
TILES = 16
NCH = 128
EPT = NCH * CH
E_PAD = TILES * EPT
N_PAD = 10112
NPT = N_PAD // TILES
LANES = 16
NBUF = 4
IW = 2 * CH


def _sc_diffuse(xflat, src_flat, w_flat, dst_flat, zrows):
    mesh = plsc.VectorSubcoreMesh(core_axis_name="c", subcore_axis_name="s")
    n_iter = NCH // NBUF

    @functools.partial(
        pl.kernel,
        out_type=jax.ShapeDtypeStruct((2 * N_PAD, H), jnp.float32),
        mesh=mesh,
        scratch_types=[
            [pltpu.VMEM((CH,), jnp.int32) for _ in range(NBUF)],
            [pltpu.VMEM((CH,), jnp.float32) for _ in range(NBUF)],
            [pltpu.VMEM((CH,), jnp.int32) for _ in range(NBUF)],
            [pltpu.VMEM((CH, H), jnp.float32) for _ in range(NBUF)],
            pltpu.VMEM_SHARED((N_PAD, H), jnp.float32),
            [pltpu.SemaphoreType.DMA for _ in range(NBUF)],
            [pltpu.SemaphoreType.DMA for _ in range(NBUF)],
            [pltpu.SemaphoreType.DMA for _ in range(NBUF)],
            [pltpu.SemaphoreType.DMA for _ in range(NBUF)],
            [pltpu.SemaphoreType.DMA for _ in range(NBUF)],
            pltpu.SemaphoreType.DMA,
        ],
    )
    def k(x_hbm, src_hbm, w_hbm, dst_hbm, z_hbm, out_hbm,
          sbufs, wbufs, dbufs, bufs, acc_sh, isems, wsems, dsems, gsems,
          ssems, zsem):
        c = lax.axis_index("c")
        s = lax.axis_index("s")
        ibase = s * EPT
        dbase = s * EPT

        def iw_dma(k_, j):
            pltpu.async_copy(src_hbm.at[pl.ds(ibase + k_ * CH, CH)],
                             sbufs[j], isems[j])
            pltpu.async_copy(w_hbm.at[pl.ds(dbase + k_ * CH, CH)],
                             wbufs[j], wsems[j])

        def iw_wait(k_, j):
            pltpu.make_async_copy(src_hbm.at[pl.ds(ibase + k_ * CH, CH)],
                                  sbufs[j], isems[j]).wait()

        def xform(j):
            for g in range(CH // LANES):
                sl = pl.ds(g * LANES, LANES)
                v = sbufs[j][sl]
                sbufs[j][sl] = v + v + c

        def w_wait(k_, j):
            pltpu.make_async_copy(w_hbm.at[pl.ds(dbase + k_ * CH, CH)],
                                  wbufs[j], wsems[j]).wait()

        def dst_dma(k_, j):
            pltpu.async_copy(dst_hbm.at[pl.ds(dbase + k_ * CH, CH)],
                             dbufs[j], dsems[j])

        def dst_wait(k_, j):
            pltpu.make_async_copy(dst_hbm.at[pl.ds(dbase + k_ * CH, CH)],
                                  dbufs[j], dsems[j]).wait()

        def gather(k_, j):
            pltpu.async_copy(x_hbm.at[sbufs[j]], bufs[j], gsems[j])

        def gather_wait(k_, j):
            pltpu.make_async_copy(x_hbm.at[sbufs[j]], bufs[j],
                                  gsems[j]).wait()

        def scatter(k_, j):
            pltpu.async_copy(bufs[j], acc_sh.at[dbufs[j]], ssems[j], add=True)

        def scatter_wait(k_, j):
            pltpu.make_async_copy(bufs[j], acc_sh.at[dbufs[j]],
                                  ssems[j]).wait()

        def multiply(j):
            buf = bufs[j]
            wb = wbufs[j]

            def grp_body(g, carry2):
                wv = wb[pl.ds(g * LANES, LANES)]
                for t in range(LANES):
                    e = g * LANES + t
                    w = wv[t]
                    for u in range(H // LANES):
                        sl = pl.ds(u * LANES, LANES)
                        buf[e, sl] = buf[e, sl] * w
                return carry2

            lax.fori_loop(0, CH // LANES, grp_body, 0)

        pltpu.async_copy(z_hbm, acc_sh.at[pl.ds(s * NPT, NPT)], zsem)

        iw_dma(0, 0)
        iw_dma(1, 1)
        dst_dma(0, 0)
        dst_dma(1, 1)
        iw_wait(0, 0)
        xform(0)
        gather(0, 0)
        iw_wait(1, 1)
        xform(1)
        gather(1, 1)
        iw_dma(2, 2)
        iw_dma(3, 3)

        pltpu.make_async_copy(z_hbm, acc_sh.at[pl.ds(s * NPT, NPT)],
                              zsem).wait()
        plsc.subcore_barrier()

        def loop_body(m, carry):
            for r in range(NBUF):
                k_ = NBUF * m + r
                jn = (r + 2) % NBUF

                def _head():
                    scatter_wait(k_ - 2, jn)

                def _pref():
                    iw_wait(k_ + 2, jn)
                    xform(jn)
                    gather(k_ + 2, jn)
                    dst_dma(k_ + 2, jn)

                if r in (0, 1):
                    @pl.when(m >= 1)
                    def _():
                        _head()

                    _pref()
                else:
                    _head()

                    @pl.when(m < n_iter - 1)
                    def _():
                        _pref()

                gather_wait(k_, r)
                w_wait(k_, r)
                multiply(r)

                @pl.when(m < n_iter - 1)
                def _():
                    iw_dma(k_ + NBUF, r)

                dst_wait(k_, r)
                scatter(k_, r)
            return carry

        lax.fori_loop(0, n_iter, loop_body, 0)
        scatter_wait(NCH - 2, (NCH - 2) % NBUF)
        scatter_wait(NCH - 1, (NCH - 1) % NBUF)
        plsc.subcore_barrier()

        pltpu.sync_copy(acc_sh.at[pl.ds(s * NPT, NPT)],
                        out_hbm.at[pl.ds(c * N_PAD + s * NPT, NPT)])

    return k(xflat, src_flat, w_flat, dst_flat, zrows)


def _mm_stats_body(al_ref, ar_ref, w1_ref, w2_ref, b_ref, z_ref, st_ref, acc):
    i = pl.program_id(0)
    z = jnp.dot(al_ref[...], w1_ref[...], preferred_element_type=jnp.float32)
    z = z + jnp.dot(ar_ref[...], w2_ref[...], preferred_element_type=jnp.float32)
    z = z + b_ref[0:1, :]
    z_ref[...] = z

    @pl.when(i == 0)
    def _():
        acc[...] = jnp.zeros((8, D), jnp.float32)

    acc[0:1, :] += jnp.sum(z, axis=0, keepdims=True)
    acc[1:2, :] += jnp.sum(z * z, axis=0, keepdims=True)

    @pl.when(i == pl.num_programs(0) - 1)
    def _():
        st_ref[...] = acc[...]


def _bn_body(z_ref, st_ref, g_ref, be_ref, o_ref):
    inv_n = 1.0 / N
    mean = st_ref[0:1, :] * inv_n
    var = st_ref[1:2, :] * inv_n - mean * mean
    inv = lax.rsqrt(var + 1e-5)
    scale = g_ref[0:1, :] * inv
    shift = be_ref[0:1, :] - mean * scale
    y = z_ref[...] * scale + shift
    o_ref[...] = jnp.where(y >= 0, y, 0.01 * y)


_BLK = 400
_GRID = N // _BLK


def _tc_finish(aggl, aggr, W, b, gamma, beta, interpret=False):
    w1 = W[:H, :]
    w2 = W[H:, :]
    b8 = jnp.broadcast_to(b.reshape(1, D), (8, D))
    g8 = jnp.broadcast_to(gamma.reshape(1, D), (8, D))
    be8 = jnp.broadcast_to(beta.reshape(1, D), (8, D))

    z, stats = pl.pallas_call(
        _mm_stats_body,
        grid=(_GRID,),
        in_specs=[
            pl.BlockSpec((_BLK, H), lambda i: (i, 0)),
            pl.BlockSpec((_BLK, H), lambda i: (i, 0)),
            pl.BlockSpec((H, D), lambda i: (0, 0)),
            pl.BlockSpec((H, D), lambda i: (0, 0)),
            pl.BlockSpec((8, D), lambda i: (0, 0)),
        ],
        out_specs=[
            pl.BlockSpec((_BLK, D), lambda i: (i, 0)),
            pl.BlockSpec((8, D), lambda i: (0, 0)),
        ],
        out_shape=[
            jax.ShapeDtypeStruct((N, D), jnp.float32),
            jax.ShapeDtypeStruct((8, D), jnp.float32),
        ],
        scratch_shapes=[pltpu.VMEM((8, D), jnp.float32)],
        interpret=interpret,
    )(aggl, aggr, w1, w2, b8)

    out = pl.pallas_call(
        _bn_body,
        grid=(_GRID,),
        in_specs=[
            pl.BlockSpec((_BLK, D), lambda i: (i, 0)),
            pl.BlockSpec((8, D), lambda i: (0, 0)),
            pl.BlockSpec((8, D), lambda i: (0, 0)),
            pl.BlockSpec((8, D), lambda i: (0, 0)),
        ],
        out_specs=pl.BlockSpec((_BLK, D), lambda i: (i, 0)),
        out_shape=jax.ShapeDtypeStruct((N, D), jnp.float32),
        interpret=interpret,
    )(z, stats, g8, be8)
    return out


def kernel(x, edge_index, edge_weight, W, b, gamma, beta):
    xflat = x.reshape(2 * N, H)
    pad = E_PAD - E
    srcp = jnp.pad(edge_index[0], (0, pad))
    dstp = jnp.pad(edge_index[1], (0, pad))
    wp = jnp.pad(edge_weight, (0, pad))
    zrows = jnp.zeros((NPT, H), jnp.float32)

    agg = _sc_diffuse(xflat, srcp, wp, dstp, zrows)
    aggl = agg[:N, :]
    aggr = agg[N_PAD:N_PAD + N, :]
    return _tc_finish(aggl, aggr, W, b, gamma, beta)
